# baseline (speedup 1.0000x reference)
import functools

import jax
import jax.numpy as jnp
from jax import lax
from jax.experimental import pallas as pl
from jax.experimental.pallas import tpu as pltpu
from jax.experimental.pallas import tpu_sc as plsc

B, P, D, N = 1024, 1024, 2048, 10
BBLK = 256
NB = B // BBLK
NC, NS = 2, 16
NW = NC * NS
RPW = B // NW


def _tc_body(ftr_ref, pk_ref, idx_ref, rsum_ref, pkn_ref):
    i = pl.program_id(0)

    @pl.when(i == 0)
    def _():
        pk = pk_ref[...]
        ss = jnp.sum(pk * pk, axis=1, keepdims=True)
        pkn_ref[...] = pk * lax.rsqrt(jnp.maximum(ss, jnp.float32(1e-12)))

    f = ftr_ref[...]
    ss = jnp.sum(f * f, axis=1, keepdims=True)
    f = f * lax.rsqrt(jnp.maximum(ss, jnp.float32(1e-12)))
    sim = lax.dot_general(f, pkn_ref[...], (((1,), (1,)), ((), ())),
                          preferred_element_type=jnp.float32)
    rmax = jnp.max(sim, axis=1, keepdims=True)
    ids = lax.broadcasted_iota(jnp.int32, sim.shape, 1)
    arg = jnp.min(jnp.where(sim == rmax, ids, jnp.int32(P)), axis=1)
    idx_ref[0, 0, :] = arg

    @pl.when(i == 0)
    def _():
        rsum_ref[0, 0] = jnp.float32(0.0)
    rsum_ref[0, 0] += jnp.sum(rmax)

    @pl.when(i == NB - 1)
    def _():
        rsum_ref[0, 0] = rsum_ref[0, 0] / jnp.float32(B)


_tc_call = pl.pallas_call(
    _tc_body,
    grid=(NB,),
    in_specs=[
        pl.BlockSpec((BBLK, D), lambda i: (i, 0)),
        pl.BlockSpec((P, D), lambda i: (0, 0)),
    ],
    out_specs=[
        pl.BlockSpec((1, 1, BBLK), lambda i: (i, 0, 0)),
        pl.BlockSpec((1, 1), lambda i: (0, 0), memory_space=pltpu.SMEM),
        pl.BlockSpec((P, D), lambda i: (0, 0)),
    ],
    out_shape=[
        jax.ShapeDtypeStruct((NB, 1, BBLK), jnp.int32),
        jax.ShapeDtypeStruct((1, 1), jnp.float32),
        jax.ShapeDtypeStruct((P, D), jnp.float32),
    ],
    compiler_params=pltpu.CompilerParams(
        dimension_semantics=("arbitrary",)),
)


CH = 2
NCHUNK = RPW // (2 * CH)


def _sc_body(prompt_hbm, pkn_hbm, idx_hbm, key_out, prom_out,
             idx_v, kbufA, kbufB, pbufA, pbufB,
             gsA, gsB, kgA, kgB, osA, osB, koA, koB):
    c = lax.axis_index("c")
    s = lax.axis_index("s")
    wid = s * NC + c
    base = wid * RPW

    pltpu.sync_copy(idx_hbm.at[wid], idx_v)

    def chunk(j, carry):
        rA = base + j * 2 * CH
        rB = rA + CH
        ivA = idx_v.at[2 * j]
        ivB = idx_v.at[2 * j + 1]
        cpA = pltpu.async_copy(prompt_hbm.at[ivA], pbufA, gsA)
        cpB = pltpu.async_copy(prompt_hbm.at[ivB], pbufB, gsB)
        ckA = pltpu.async_copy(pkn_hbm.at[ivA], kbufA, kgA)
        ckB = pltpu.async_copy(pkn_hbm.at[ivB], kbufB, kgB)
        cpA.wait()
        oA = pltpu.async_copy(pbufA, prom_out.at[pl.ds(rA, CH)], osA)
        ckA.wait()
        okA = pltpu.async_copy(kbufA, key_out.at[pl.ds(rA, CH)], koA)
        cpB.wait()
        oB = pltpu.async_copy(pbufB, prom_out.at[pl.ds(rB, CH)], osB)
        ckB.wait()
        okB = pltpu.async_copy(kbufB, key_out.at[pl.ds(rB, CH)], koB)
        oA.wait()
        okA.wait()
        oB.wait()
        okB.wait()
        return carry

    lax.fori_loop(0, NCHUNK, chunk, jnp.int32(0))


def _sc_call(prompt2d, pkn, idx3d):
    mesh = plsc.VectorSubcoreMesh(core_axis_name="c", subcore_axis_name="s",
                                  num_cores=NC, num_subcores=NS)
    f = pl.kernel(
        _sc_body,
        out_type=[
            jax.ShapeDtypeStruct((B, D), jnp.float32),
            jax.ShapeDtypeStruct((B, N * D), jnp.float32),
        ],
        mesh=mesh,
        scratch_types=[
            pltpu.VMEM((RPW // CH, CH), jnp.int32),
            pltpu.VMEM((CH, D), jnp.float32),
            pltpu.VMEM((CH, D), jnp.float32),
            pltpu.VMEM((CH, N * D), jnp.float32),
            pltpu.VMEM((CH, N * D), jnp.float32),
        ] + [pltpu.SemaphoreType.DMA] * 8,
    )
    return f(prompt2d, pkn, idx3d)


def kernel(ftr, prompt, prompt_key):
    idx_blk, rsum, pkn = _tc_call(ftr, prompt_key)
    idx_flat = idx_blk.reshape(B)
    key2d, prom2d = _sc_call(prompt.reshape(P, N * D), pkn,
                             idx_flat.reshape(NW, RPW // CH, CH))
    return (idx_flat.reshape(B, 1),
            key2d.reshape(B, 1, D),
            rsum[0, 0],
            prom2d.reshape(B, 1, N, D))


# speedup vs baseline: 1.2701x; 1.2701x over previous
import functools

import jax
import jax.numpy as jnp
from jax import lax
from jax.experimental import pallas as pl
from jax.experimental.pallas import tpu as pltpu
from jax.experimental.pallas import tpu_sc as plsc

B, P, D, N = 1024, 1024, 2048, 10
BBLK = 256
NB = B // BBLK
NC, NS = 2, 16
NW = NC * NS
RPW = B // NW


def _tc_body(ftr_ref, pk_ref, idx_ref, rsum_ref, pkn_ref):
    i = pl.program_id(0)

    @pl.when(i == 0)
    def _():
        pk = pk_ref[...]
        ss = jnp.sum(pk * pk, axis=1, keepdims=True)
        pkn_ref[...] = pk * lax.rsqrt(jnp.maximum(ss, jnp.float32(1e-12)))

    f = ftr_ref[...]
    ss = jnp.sum(f * f, axis=1, keepdims=True)
    f = f * lax.rsqrt(jnp.maximum(ss, jnp.float32(1e-12)))
    sim = lax.dot_general(f, pkn_ref[...], (((1,), (1,)), ((), ())),
                          preferred_element_type=jnp.float32)
    rmax = jnp.max(sim, axis=1, keepdims=True)
    ids = lax.broadcasted_iota(jnp.int32, sim.shape, 1)
    arg = jnp.min(jnp.where(sim == rmax, ids, jnp.int32(P)), axis=1)
    idx_ref[0, 0, :] = arg

    @pl.when(i == 0)
    def _():
        rsum_ref[0, 0] = jnp.float32(0.0)
    rsum_ref[0, 0] += jnp.sum(rmax)

    @pl.when(i == NB - 1)
    def _():
        rsum_ref[0, 0] = rsum_ref[0, 0] / jnp.float32(B)


_tc_call = pl.pallas_call(
    _tc_body,
    grid=(NB,),
    in_specs=[
        pl.BlockSpec((BBLK, D), lambda i: (i, 0)),
        pl.BlockSpec((P, D), lambda i: (0, 0)),
    ],
    out_specs=[
        pl.BlockSpec((1, 1, BBLK), lambda i: (i, 0, 0)),
        pl.BlockSpec((1, 1), lambda i: (0, 0), memory_space=pltpu.SMEM),
        pl.BlockSpec((P, D), lambda i: (0, 0)),
    ],
    out_shape=[
        jax.ShapeDtypeStruct((NB, 1, BBLK), jnp.int32),
        jax.ShapeDtypeStruct((1, 1), jnp.float32),
        jax.ShapeDtypeStruct((P, D), jnp.float32),
    ],
    compiler_params=pltpu.CompilerParams(
        dimension_semantics=("arbitrary",)),
)


CH = 2
NCHUNK = RPW // (2 * CH)


def _sc_keys_body(pkn_hbm, idx_hbm, key_out,
                  idx_v, kbufA, kbufB, kgA, kgB, koA, koB):
    c = lax.axis_index("c")
    s = lax.axis_index("s")
    wid = s * NC + c
    base = wid * RPW

    pltpu.sync_copy(idx_hbm.at[wid], idx_v)

    def chunk(j, carry):
        rA = base + j * 2 * CH
        rB = rA + CH
        ivA = idx_v.at[2 * j]
        ivB = idx_v.at[2 * j + 1]
        ckA = pltpu.async_copy(pkn_hbm.at[ivA], kbufA, kgA)
        ckB = pltpu.async_copy(pkn_hbm.at[ivB], kbufB, kgB)
        ckA.wait()
        okA = pltpu.async_copy(kbufA, key_out.at[pl.ds(rA, CH)], koA)
        ckB.wait()
        okB = pltpu.async_copy(kbufB, key_out.at[pl.ds(rB, CH)], koB)
        okA.wait()
        okB.wait()
        return carry

    lax.fori_loop(0, NCHUNK, chunk, jnp.int32(0))


def _sc_keys_call(pkn, idx3d):
    mesh = plsc.VectorSubcoreMesh(core_axis_name="c", subcore_axis_name="s",
                                  num_cores=NC, num_subcores=NS)
    f = pl.kernel(
        _sc_keys_body,
        out_type=jax.ShapeDtypeStruct((B, D), jnp.float32),
        mesh=mesh,
        scratch_types=[
            pltpu.VMEM((RPW // CH, CH), jnp.int32),
            pltpu.VMEM((CH, D), jnp.float32),
            pltpu.VMEM((CH, D), jnp.float32),
        ] + [pltpu.SemaphoreType.DMA] * 4,
    )
    return f(pkn, idx3d)


def _sc_prompt_body(prompt_hbm, idx_hbm, prom_out,
                    idx_v, pbufA, pbufB, gsA, gsB, osA, osB):
    c = lax.axis_index("c")
    s = lax.axis_index("s")
    wid = s * NC + c
    base = wid * RPW

    pltpu.sync_copy(idx_hbm.at[pl.ds(base, RPW)], idx_v)

    bufs = (pbufA, pbufB)
    gsems = (gsA, gsB)
    osems = (osA, osB)
    ivecs = [idx_v[pl.ds(k * 16, 16)] for k in range(RPW // 16)]
    outcp = [None, None]
    for g in range(RPW):
        b = g % 2
        if outcp[b] is not None:
            outcp[b].wait()
        iv = ivecs[g // 16][g % 16]
        cp = pltpu.async_copy(prompt_hbm.at[pl.ds(iv, 1)], bufs[b], gsems[b])
        cp.wait()
        outcp[b] = pltpu.async_copy(bufs[b], prom_out.at[pl.ds(base + g, 1)],
                                    osems[b])
    outcp[0].wait()
    outcp[1].wait()


def _sc_prompt_call(prompt, idx):
    mesh = plsc.VectorSubcoreMesh(core_axis_name="c", subcore_axis_name="s",
                                  num_cores=NC, num_subcores=NS)
    f = pl.kernel(
        _sc_prompt_body,
        out_type=jax.ShapeDtypeStruct((B, N, D), jnp.float32),
        mesh=mesh,
        scratch_types=[
            pltpu.VMEM((RPW,), jnp.int32),
            pltpu.VMEM((1, N, D), jnp.float32),
            pltpu.VMEM((1, N, D), jnp.float32),
        ] + [pltpu.SemaphoreType.DMA] * 4,
        compiler_params=pltpu.CompilerParams(use_tc_tiling_on_sc=True),
    )
    return f(prompt, idx)


def kernel(ftr, prompt, prompt_key):
    idx_blk, rsum, pkn = _tc_call(ftr, prompt_key)
    idx_flat = idx_blk.reshape(B)
    key2d = _sc_keys_call(pkn, idx_flat.reshape(NW, RPW // CH, CH))
    prom3d = _sc_prompt_call(prompt, idx_flat)
    return (idx_flat.reshape(B, 1),
            key2d.reshape(B, 1, D),
            rsum[0, 0],
            prom3d.reshape(B, 1, N, D))
